# Initial kernel scaffold; baseline (speedup 1.0000x reference)
import jax
import jax.numpy as jnp
from jax.experimental import pallas as pl


def kernel(query, pos_items, c0, c1, c0_, c1_, wkk, indices, indptr, cd0, cd1):
    raise NotImplementedError("write your pallas kernel here")


# speedup vs baseline: 8.3608x; 8.3608x over previous
import functools

import jax
import jax.numpy as jnp
import numpy as np
from jax import lax
from jax.experimental import pallas as pl
from jax.experimental.pallas import tpu as pltpu
from jax.experimental.pallas import tpu_sc as plsc

B = 4096
D = 64
K = 64
NN = 64
L = 20


_ROTS = ((13, 15, 26, 6), (17, 29, 16, 24))


def _np_threefry(k0, k1, x0, x1):
    def rotl(x, d):
        return ((x << np.uint32(d)) | (x >> np.uint32(32 - d))).astype(np.uint32)

    k0 = np.uint32(k0)
    k1 = np.uint32(k1)
    ks = [k0, k1, np.uint32(k0 ^ k1 ^ np.uint32(0x1BD11BDA))]
    x0 = (x0 + k0).astype(np.uint32)
    x1 = (x1 + k1).astype(np.uint32)
    for r in range(5):
        for rot in _ROTS[r % 2]:
            x0 = (x0 + x1).astype(np.uint32)
            x1 = rotl(x1, rot)
            x1 = (x1 ^ x0).astype(np.uint32)
        x0 = (x0 + ks[(r + 1) % 3]).astype(np.uint32)
        x1 = (x1 + ks[(r + 2) % 3] + np.uint32(r + 1)).astype(np.uint32)
    return x0, x1


def _fold(key, data):
    o0, o1 = _np_threefry(key[0], key[1], np.uint32([0]), np.uint32([data]))
    return (int(o0[0]), int(o1[0]))


_BASE_KEY = (0, 42)
_KEY_G0 = _fold(_BASE_KEY, 0)
_KEY_G1 = _fold(_BASE_KEY, 1)
_KEY_U = _fold(_BASE_KEY, 2)

_TINY = np.float32(np.finfo(np.float32).tiny)


def _i32(x):
    return np.int32(np.uint32(x & 0xFFFFFFFF).view(np.int32))


def _tf_bits(key, ctr):
    k0 = _i32(key[0])
    k1 = _i32(key[1])
    ks2 = _i32(key[0] ^ key[1] ^ 0x1BD11BDA)
    ks = [k0, k1, ks2]

    def rotl(x, d):
        return lax.shift_left(x, np.int32(d)) | lax.shift_right_logical(
            x, np.int32(32 - d)
        )

    x0 = jnp.full_like(ctr, k0)
    x1 = ctr + k1
    for r in range(5):
        for rot in _ROTS[r % 2]:
            x0 = x0 + x1
            x1 = rotl(x1, rot)
            x1 = x1 ^ x0
        x0 = x0 + ks[(r + 1) % 3]
        x1 = x1 + ks[(r + 2) % 3] + np.int32(r + 1)
    return x0 ^ x1


def _bits_to_u01(bits):
    fb = lax.shift_right_logical(bits, np.int32(9)) | np.int32(0x3F800000)
    return lax.bitcast_convert_type(fb, jnp.float32) - np.float32(1.0)


def _gumbel_from_bits(bits):
    u = _bits_to_u01(bits)
    u = jnp.maximum(_TINY, u * np.float32(np.float32(1.0) - _TINY) + _TINY)
    return -jnp.log(-jnp.log(u))


R = 64
NLANES = R * NN
GRID = B // R


def _dot(a, b, dims, precision=None):
    return lax.dot_general(a, b, (dims, ((), ())),
                           preferred_element_type=jnp.float32,
                           precision=precision)


def _dot_x(a, b, dims):
    return _dot(a, b, dims, precision=lax.Precision.HIGHEST)


def _tc_body(q_ref, c0_ref, c1_ref, c0p_ref, c1p_ref, wkk_ref,
             k01_ref, u_ref, negp_ref, w0_ref, w1_ref):
    i = pl.program_id(0)
    q = q_ref[...]
    q0 = q[:, :32]
    q1 = q[:, 32:]
    r0 = _dot(q0, c0_ref[...], ((1,), (1,)))
    r1 = _dot(q1, c1_ref[...], ((1,), (1,)))

    def softmax(x):
        m = jnp.max(x, axis=-1, keepdims=True)
        e = jnp.exp(x - m)
        return e / jnp.sum(e, axis=-1, keepdims=True)

    r0s = softmax(r0)
    r1s = softmax(r1)
    wkk = wkk_ref[...]
    s0 = _dot(r1s, wkk, ((1,), (1,))) * r0s
    logits0 = jnp.where(s0 > 0, jnp.log(jnp.maximum(s0, 1e-30)), -1e30)
    logits0T = logits0.T
    r0T = r0.T
    r1T = r1.T
    r1sT = r1s.T

    lane_b = lax.shift_right_logical(
        lax.broadcasted_iota(jnp.int32, (R, NLANES), 1), np.int32(6)
    )
    row_r = lax.broadcasted_iota(jnp.int32, (R, NLANES), 0)
    E = jnp.where(lane_b == row_r, np.float32(1.0), np.float32(0.0))

    Ltile = _dot_x(logits0T, E, ((1,), (0,)))
    r1sTile = _dot_x(r1sT, E, ((1,), (0,)))
    r0Tile = _dot_x(r0T, E, ((1,), (0,)))
    r1Tile = _dot_x(r1T, E, ((1,), (0,)))

    lane = lax.broadcasted_iota(jnp.int32, (K, NLANES), 1)
    kio = lax.broadcasted_iota(jnp.int32, (K, NLANES), 0)
    ctr = i * np.int32(NLANES * K) + lane * np.int32(K) + kio

    g0 = _gumbel_from_bits(_tf_bits(_KEY_G0, ctr))
    t0 = Ltile + g0
    m0 = jnp.max(t0, axis=0, keepdims=True)
    k0 = jnp.min(jnp.where(t0 == m0, kio, np.int32(K)), axis=0, keepdims=True)

    oh0 = jnp.where(kio == k0, np.float32(1.0), np.float32(0.0))
    subwkkT = _dot_x(wkk, oh0, ((0,), (0,)))
    s1T = subwkkT * r1sTile
    g1 = _gumbel_from_bits(_tf_bits(_KEY_G1, ctr))
    t1 = jnp.where(s1T > 0, jnp.log(jnp.maximum(s1T, 1e-30)), -1e30) + g1
    m1 = jnp.max(t1, axis=0, keepdims=True)
    k1 = jnp.min(jnp.where(t1 == m1, kio, np.int32(K)), axis=0, keepdims=True)
    oh1 = jnp.where(kio == k1, np.float32(1.0), np.float32(0.0))

    p0 = jnp.sum(oh0 * r0Tile, axis=0, keepdims=True)
    p1 = jnp.sum(oh1 * r1Tile, axis=0, keepdims=True)
    negp_ref[...] = p0 + p1
    k01_ref[...] = k0 * np.int32(K) + k1

    ctr2 = i * np.int32(NLANES) + lax.broadcasted_iota(jnp.int32, (1, NLANES), 1)
    u01 = _bits_to_u01(_tf_bits(_KEY_U, ctr2))
    u_ref[...] = jnp.maximum(np.float32(0.0), u01)

    w0_ref[...] = _dot(q0, c0p_ref[...], ((1,), (1,)))
    w1_ref[...] = _dot(q1, c1p_ref[...], ((1,), (1,)))


def _tc_sample(query, c0, c1, c0_, c1_, wkk):
    full = lambda s: pl.BlockSpec(s, lambda i: (0, 0))
    return pl.pallas_call(
        _tc_body,
        grid=(GRID,),
        in_specs=[
            pl.BlockSpec((R, D), lambda i: (i, 0)),
            full((K, 32)),
            full((K, 32)),
            full((K + 1, 32)),
            full((K + 1, 32)),
            full((K, K)),
        ],
        out_specs=[
            pl.BlockSpec((1, NLANES), lambda i: (0, i)),
            pl.BlockSpec((1, NLANES), lambda i: (0, i)),
            pl.BlockSpec((1, NLANES), lambda i: (0, i)),
            pl.BlockSpec((R, K + 1), lambda i: (i, 0)),
            pl.BlockSpec((R, K + 1), lambda i: (i, 0)),
        ],
        out_shape=[
            jax.ShapeDtypeStruct((1, B * NN), jnp.int32),
            jax.ShapeDtypeStruct((1, B * NN), jnp.float32),
            jax.ShapeDtypeStruct((1, B * NN), jnp.float32),
            jax.ShapeDtypeStruct((B, K + 1), jnp.float32),
            jax.ShapeDtypeStruct((B, K + 1), jnp.float32),
        ],
    )(query, c0, c1, c0_, c1_, wkk)


NC = 2
NS = 16
NW = NC * NS
NEG_T = B * NN
POS_T = B * L
NEG_W = NEG_T // NW
POS_W = POS_T // NW
CH = 128
NEG_CH = NEG_W // CH
POS_CH = POS_W // CH
VPC = CH // 16


def _sc_body(k01_hbm, u_hbm, iptr_hbm, items_hbm, pos_hbm, cd0_hbm, cd1_hbm,
             w0_hbm, w1_hbm,
             neg_out, pp_out,
             k01_v, u_v, iptr_v, idx_v, rows_v, pi_v, kp_v, fi_v, wr_v, pp_v,
             sem):
    wid = lax.axis_index("s") * NC + lax.axis_index("c")
    nbase = wid * NEG_W

    pltpu.sync_copy(k01_hbm.at[pl.ds(nbase, NEG_W)], k01_v)
    pltpu.sync_copy(u_hbm.at[pl.ds(nbase, NEG_W)], u_v)
    pltpu.sync_copy(iptr_hbm, iptr_v)

    def neg_pos_body(j, carry):
        sl = pl.ds(j * 16, 16)
        idx = k01_v[sl]
        s = plsc.load_gather(iptr_v, [idx])
        e = plsc.load_gather(iptr_v, [idx + 1])
        ii = ((e - s).astype(jnp.float32) * u_v[sl]).astype(jnp.int32)
        idx_v[j // VPC, pl.ds((j % VPC) * 16, 16)] = ii + s
        return carry

    lax.fori_loop(0, NEG_W // 16, neg_pos_body, 0)

    def neg_fire(j, carry):
        pltpu.async_copy(items_hbm.at[idx_v.at[j]], rows_v.at[pl.ds(j * CH, CH)], sem)
        return carry

    lax.fori_loop(0, NEG_CH, neg_fire, 0)
    pltpu.make_async_copy(items_hbm.at[pl.ds(0, NEG_W)], rows_v, sem).wait()

    def neg_add1(j, carry):
        sl = pl.ds(j * 16, 16)
        rows_v[sl] = rows_v[sl] + 1
        return carry

    lax.fori_loop(0, NEG_W // 16, neg_add1, 0)
    pltpu.sync_copy(rows_v, neg_out.at[pl.ds(nbase, NEG_W)])

    pbase = wid * POS_W
    pltpu.sync_copy(pos_hbm.at[wid], pi_v)

    def cd_gather(cd_hbm):
        def fire(j, carry):
            pltpu.async_copy(cd_hbm.at[pi_v.at[j]], kp_v.at[pl.ds(j * CH, CH)], sem)
            return carry
        lax.fori_loop(0, POS_CH, fire, 0)
        pltpu.make_async_copy(cd_hbm.at[pl.ds(0, POS_W)], kp_v, sem).wait()

    def w_gather(w_hbm):
        def fire(j, carry):
            pltpu.async_copy(w_hbm.at[fi_v.at[j]], wr_v.at[pl.ds(j * CH, CH)], sem)
            return carry
        lax.fori_loop(0, POS_CH, fire, 0)
        pltpu.make_async_copy(w_hbm.at[pl.ds(0, POS_W)], wr_v, sem).wait()

    def make_fi(j, carry):
        sl = pl.ds(j * 16, 16)
        gi = pbase + j * 16 + lax.iota(jnp.int32, 16)
        b = lax.div(gi, np.int32(L))
        fi_v[j // VPC, pl.ds((j % VPC) * 16, 16)] = b * np.int32(K + 1) + kp_v[sl]
        return carry

    def pp_set(j, carry):
        sl = pl.ds(j * 16, 16)
        pp_v[sl] = wr_v[sl]
        return carry

    def pp_add(j, carry):
        sl = pl.ds(j * 16, 16)
        pp_v[sl] = pp_v[sl] + wr_v[sl]
        return carry

    cd_gather(cd0_hbm)
    lax.fori_loop(0, POS_W // 16, make_fi, 0)
    w_gather(w0_hbm)
    lax.fori_loop(0, POS_W // 16, pp_set, 0)
    cd_gather(cd1_hbm)
    lax.fori_loop(0, POS_W // 16, make_fi, 0)
    w_gather(w1_hbm)
    lax.fori_loop(0, POS_W // 16, pp_add, 0)
    pltpu.sync_copy(pp_v, pp_out.at[pl.ds(pbase, POS_W)])


def _sc_gather(k01_flat, u_flat, indptr, indices, pos_flat, cd0, cd1, w0f, w1f):
    mesh = plsc.VectorSubcoreMesh(core_axis_name="c", subcore_axis_name="s")
    fn = pl.kernel(
        _sc_body,
        out_type=(
            jax.ShapeDtypeStruct((NEG_T,), jnp.int32),
            jax.ShapeDtypeStruct((POS_T,), jnp.float32),
        ),
        mesh=mesh,
        scratch_types=[
            pltpu.VMEM((NEG_W,), jnp.int32),
            pltpu.VMEM((NEG_W,), jnp.float32),
            pltpu.VMEM((K * K + 1,), jnp.int32),
            pltpu.VMEM((NEG_CH, CH), jnp.int32),
            pltpu.VMEM((NEG_W,), jnp.int32),
            pltpu.VMEM((POS_CH, CH), jnp.int32),
            pltpu.VMEM((POS_W,), jnp.int32),
            pltpu.VMEM((POS_CH, CH), jnp.int32),
            pltpu.VMEM((POS_W,), jnp.float32),
            pltpu.VMEM((POS_W,), jnp.float32),
            pltpu.SemaphoreType.DMA,
        ],
        compiler_params=pltpu.CompilerParams(needs_layout_passes=False),
    )
    return fn(k01_flat, u_flat, indptr, indices, pos_flat, cd0, cd1, w0f, w1f)


def kernel(query, pos_items, c0, c1, c0_, c1_, wkk, indices, indptr, cd0, cd1):
    k01, u, negp, w0, w1 = _tc_sample(query, c0, c1, c0_, c1_, wkk)
    w0f = w0.reshape(-1)
    w1f = w1.reshape(-1)
    neg_flat, pp_flat = _sc_gather(
        k01.reshape(-1),
        u.reshape(-1),
        indptr.astype(jnp.int32),
        indices.astype(jnp.int32),
        pos_items.reshape(NW, POS_CH, CH).astype(jnp.int32),
        cd0.astype(jnp.int32),
        cd1.astype(jnp.int32),
        w0f,
        w1f,
    )
    pos_prob = pp_flat.reshape(B, L)
    neg_items = neg_flat.reshape(B, NN)
    neg_prob = negp.reshape(B, NN)
    return (pos_prob, neg_items, neg_prob)


# speedup vs baseline: 9.6645x; 1.1559x over previous
import functools

import jax
import jax.numpy as jnp
import numpy as np
from jax import lax
from jax.experimental import pallas as pl
from jax.experimental.pallas import tpu as pltpu
from jax.experimental.pallas import tpu_sc as plsc

B = 4096
D = 64
K = 64
NN = 64
L = 20


_ROTS = ((13, 15, 26, 6), (17, 29, 16, 24))


def _np_threefry(k0, k1, x0, x1):
    def rotl(x, d):
        return ((x << np.uint32(d)) | (x >> np.uint32(32 - d))).astype(np.uint32)

    k0 = np.uint32(k0)
    k1 = np.uint32(k1)
    ks = [k0, k1, np.uint32(k0 ^ k1 ^ np.uint32(0x1BD11BDA))]
    x0 = (x0 + k0).astype(np.uint32)
    x1 = (x1 + k1).astype(np.uint32)
    for r in range(5):
        for rot in _ROTS[r % 2]:
            x0 = (x0 + x1).astype(np.uint32)
            x1 = rotl(x1, rot)
            x1 = (x1 ^ x0).astype(np.uint32)
        x0 = (x0 + ks[(r + 1) % 3]).astype(np.uint32)
        x1 = (x1 + ks[(r + 2) % 3] + np.uint32(r + 1)).astype(np.uint32)
    return x0, x1


def _fold(key, data):
    o0, o1 = _np_threefry(key[0], key[1], np.uint32([0]), np.uint32([data]))
    return (int(o0[0]), int(o1[0]))


_BASE_KEY = (0, 42)
_KEY_G0 = _fold(_BASE_KEY, 0)
_KEY_G1 = _fold(_BASE_KEY, 1)
_KEY_U = _fold(_BASE_KEY, 2)

_TINY = np.float32(np.finfo(np.float32).tiny)


def _i32(x):
    return np.int32(np.uint32(x & 0xFFFFFFFF).view(np.int32))


def _tf_bits(key, ctr):
    k0 = _i32(key[0])
    k1 = _i32(key[1])
    ks2 = _i32(key[0] ^ key[1] ^ 0x1BD11BDA)
    ks = [k0, k1, ks2]

    def rotl(x, d):
        return lax.shift_left(x, np.int32(d)) | lax.shift_right_logical(
            x, np.int32(32 - d)
        )

    x0 = jnp.full_like(ctr, k0)
    x1 = ctr + k1
    for r in range(5):
        for rot in _ROTS[r % 2]:
            x0 = x0 + x1
            x1 = rotl(x1, rot)
            x1 = x1 ^ x0
        x0 = x0 + ks[(r + 1) % 3]
        x1 = x1 + ks[(r + 2) % 3] + np.int32(r + 1)
    return x0 ^ x1


def _bits_to_u01(bits):
    fb = lax.shift_right_logical(bits, np.int32(9)) | np.int32(0x3F800000)
    return lax.bitcast_convert_type(fb, jnp.float32) - np.float32(1.0)


def _gumbel_from_bits(bits):
    u = _bits_to_u01(bits)
    u = jnp.maximum(_TINY, u * np.float32(np.float32(1.0) - _TINY) + _TINY)
    return -jnp.log(-jnp.log(u))


R = 128
NLANES = R * NN
GRID = B // R


def _dot(a, b, dims, precision=None):
    return lax.dot_general(a, b, (dims, ((), ())),
                           preferred_element_type=jnp.float32,
                           precision=precision)


def _dot_x(a, b, dims):
    return _dot(a, b, dims, precision=lax.Precision.HIGHEST)


def _tc_body(q_ref, c0_ref, c1_ref, c0p_ref, c1p_ref, wkk_ref,
             k01_ref, u_ref, rr_ref, w0_ref, w1_ref):
    i = pl.program_id(0)
    q = q_ref[...]
    q0 = q[:, :32]
    q1 = q[:, 32:]
    r0 = _dot(q0, c0_ref[...], ((1,), (1,)))
    r1 = _dot(q1, c1_ref[...], ((1,), (1,)))

    def softmax(x):
        m = jnp.max(x, axis=-1, keepdims=True)
        e = jnp.exp(x - m)
        return e / jnp.sum(e, axis=-1, keepdims=True)

    r0s = softmax(r0)
    r1s = softmax(r1)
    wkk = wkk_ref[...]
    s0 = _dot(r1s, wkk, ((1,), (1,))) * r0s
    logits0 = jnp.where(s0 > 0, jnp.log(jnp.maximum(s0, 1e-30)), -1e30)
    logits0T = logits0.T
    r1sT = r1s.T
    rr_ref[...] = jnp.concatenate([r0, r1], axis=1)

    lane_b = lax.shift_right_logical(
        lax.broadcasted_iota(jnp.int32, (R, NLANES), 1), np.int32(6)
    )
    row_r = lax.broadcasted_iota(jnp.int32, (R, NLANES), 0)
    E = jnp.where(lane_b == row_r, np.float32(1.0), np.float32(0.0))

    Ltile = _dot_x(logits0T, E, ((1,), (0,)))
    r1sTile = _dot_x(r1sT, E, ((1,), (0,)))

    lane = lax.broadcasted_iota(jnp.int32, (K, NLANES), 1)
    kio = lax.broadcasted_iota(jnp.int32, (K, NLANES), 0)
    ctr = i * np.int32(NLANES * K) + lane * np.int32(K) + kio

    g0 = _gumbel_from_bits(_tf_bits(_KEY_G0, ctr))
    t0 = Ltile + g0
    m0 = jnp.max(t0, axis=0, keepdims=True)
    k0 = jnp.min(jnp.where(t0 == m0, kio, np.int32(K)), axis=0, keepdims=True)

    oh0 = jnp.where(kio == k0, np.float32(1.0), np.float32(0.0))
    subwkkT = _dot_x(wkk, oh0, ((0,), (0,)))
    s1T = subwkkT * r1sTile
    g1 = _gumbel_from_bits(_tf_bits(_KEY_G1, ctr))
    t1 = jnp.where(s1T > 0, jnp.log(jnp.maximum(s1T, 1e-30)), -1e30) + g1
    m1 = jnp.max(t1, axis=0, keepdims=True)
    k1 = jnp.min(jnp.where(t1 == m1, kio, np.int32(K)), axis=0, keepdims=True)
    k01_ref[...] = k0 * np.int32(K) + k1

    ctr2 = i * np.int32(NLANES) + lax.broadcasted_iota(jnp.int32, (1, NLANES), 1)
    u01 = _bits_to_u01(_tf_bits(_KEY_U, ctr2))
    u_ref[...] = jnp.maximum(np.float32(0.0), u01)

    w0_ref[...] = _dot(q0, c0p_ref[...], ((1,), (1,)))
    w1_ref[...] = _dot(q1, c1p_ref[...], ((1,), (1,)))


def _tc_sample(query, c0, c1, c0_, c1_, wkk):
    full = lambda s: pl.BlockSpec(s, lambda i: (0, 0))
    return pl.pallas_call(
        _tc_body,
        grid=(GRID,),
        in_specs=[
            pl.BlockSpec((R, D), lambda i: (i, 0)),
            full((K, 32)),
            full((K, 32)),
            full((2 * K, 32)),
            full((2 * K, 32)),
            full((K, K)),
        ],
        out_specs=[
            pl.BlockSpec((1, NLANES), lambda i: (0, i)),
            pl.BlockSpec((1, NLANES), lambda i: (0, i)),
            pl.BlockSpec((R, 2 * K), lambda i: (i, 0)),
            pl.BlockSpec((R, 2 * K), lambda i: (i, 0)),
            pl.BlockSpec((R, 2 * K), lambda i: (i, 0)),
        ],
        out_shape=[
            jax.ShapeDtypeStruct((1, B * NN), jnp.int32),
            jax.ShapeDtypeStruct((1, B * NN), jnp.float32),
            jax.ShapeDtypeStruct((B, 2 * K), jnp.float32),
            jax.ShapeDtypeStruct((B, 2 * K), jnp.float32),
            jax.ShapeDtypeStruct((B, 2 * K), jnp.float32),
        ],
    )(query, c0, c1,
      jnp.pad(c0_, ((0, K - 1), (0, 0))),
      jnp.pad(c1_, ((0, K - 1), (0, 0))),
      wkk)


NC = 2
NS = 16
NW = NC * NS
NEG_T = B * NN
POS_T = B * L
NEG_W = NEG_T // NW
POS_W = POS_T // NW
CH = 128
NEG_CH = NEG_W // CH
POS_CH = POS_W // CH
VPC = CH // 16


def _sc_body(k01_hbm, u_hbm, iptr_hbm, items_hbm, pos_hbm, cd0_hbm, cd1_hbm,
             w0_hbm, w1_hbm, rr_hbm,
             neg_out, negp_out, pp_out,
             k01_v, u_v, iptr_v, idx_v, rows_v, p0r_v, p1r_v,
             pi_v, kp_v, fi_v, wr_v, pp_v,
             sem):
    wid = lax.axis_index("s") * NC + lax.axis_index("c")
    nbase = wid * NEG_W

    pltpu.sync_copy(k01_hbm.at[pl.ds(nbase, NEG_W)], k01_v)
    pltpu.sync_copy(u_hbm.at[pl.ds(nbase, NEG_W)], u_v)
    pltpu.sync_copy(iptr_hbm, iptr_v)

    def neg_pos_body(j, carry):
        sl = pl.ds(j * 16, 16)
        idx = k01_v[sl]
        s = plsc.load_gather(iptr_v, [idx])
        e = plsc.load_gather(iptr_v, [idx + 1])
        ii = ((e - s).astype(jnp.float32) * u_v[sl]).astype(jnp.int32)
        idx_v[j // VPC, pl.ds((j % VPC) * 16, 16)] = ii + s
        return carry

    lax.fori_loop(0, NEG_W // 16, neg_pos_body, 0)

    def neg_fire(j, carry):
        pltpu.async_copy(items_hbm.at[idx_v.at[j]], rows_v.at[pl.ds(j * CH, CH)], sem)
        return carry

    lax.fori_loop(0, NEG_CH, neg_fire, 0)
    pltpu.make_async_copy(items_hbm.at[pl.ds(0, NEG_W)], rows_v, sem).wait()

    def neg_add1(j, carry):
        sl = pl.ds(j * 16, 16)
        rows_v[sl] = rows_v[sl] + 1
        return carry

    lax.fori_loop(0, NEG_W // 16, neg_add1, 0)
    pltpu.sync_copy(rows_v, neg_out.at[pl.ds(nbase, NEG_W)])

    def fi0_body(j, carry):
        sl = pl.ds(j * 16, 16)
        bn = nbase + j * 16 + lax.iota(jnp.int32, 16)
        b = lax.shift_right_logical(bn, 6)
        k0 = lax.shift_right_logical(k01_v[sl], 6)
        idx_v[j // VPC, pl.ds((j % VPC) * 16, 16)] = b * np.int32(128) + k0
        return carry

    def fi1_body(j, carry):
        sl = pl.ds(j * 16, 16)
        bn = nbase + j * 16 + lax.iota(jnp.int32, 16)
        b = lax.shift_right_logical(bn, 6)
        k1 = k01_v[sl] & np.int32(63)
        idx_v[j // VPC, pl.ds((j % VPC) * 16, 16)] = b * np.int32(128) + np.int32(K) + k1
        return carry

    def rr_fire(dst_v):
        def fire(j, carry):
            pltpu.async_copy(rr_hbm.at[idx_v.at[j]], dst_v.at[pl.ds(j * CH, CH)], sem)
            return carry
        lax.fori_loop(0, NEG_CH, fire, 0)
        pltpu.make_async_copy(rr_hbm.at[pl.ds(0, NEG_W)], dst_v, sem).wait()

    lax.fori_loop(0, NEG_W // 16, fi0_body, 0)
    rr_fire(p0r_v)
    lax.fori_loop(0, NEG_W // 16, fi1_body, 0)
    rr_fire(p1r_v)

    def negp_body(j, carry):
        sl = pl.ds(j * 16, 16)
        p0r_v[sl] = p0r_v[sl] + p1r_v[sl]
        return carry

    lax.fori_loop(0, NEG_W // 16, negp_body, 0)
    pltpu.sync_copy(p0r_v, negp_out.at[pl.ds(nbase, NEG_W)])

    pbase = wid * POS_W
    pltpu.sync_copy(pos_hbm.at[wid], pi_v)

    def cd_gather(cd_hbm):
        def fire(j, carry):
            pltpu.async_copy(cd_hbm.at[pi_v.at[j]], kp_v.at[pl.ds(j * CH, CH)], sem)
            return carry
        lax.fori_loop(0, POS_CH, fire, 0)
        pltpu.make_async_copy(cd_hbm.at[pl.ds(0, POS_W)], kp_v, sem).wait()

    def w_gather(w_hbm):
        def fire(j, carry):
            pltpu.async_copy(w_hbm.at[fi_v.at[j]], wr_v.at[pl.ds(j * CH, CH)], sem)
            return carry
        lax.fori_loop(0, POS_CH, fire, 0)
        pltpu.make_async_copy(w_hbm.at[pl.ds(0, POS_W)], wr_v, sem).wait()

    def make_fi(j, carry):
        sl = pl.ds(j * 16, 16)
        gi = pbase + j * 16 + lax.iota(jnp.int32, 16)
        b = lax.div(gi, np.int32(L))
        fi_v[j // VPC, pl.ds((j % VPC) * 16, 16)] = b * np.int32(128) + kp_v[sl]
        return carry

    def pp_set(j, carry):
        sl = pl.ds(j * 16, 16)
        pp_v[sl] = wr_v[sl]
        return carry

    def pp_add(j, carry):
        sl = pl.ds(j * 16, 16)
        pp_v[sl] = pp_v[sl] + wr_v[sl]
        return carry

    cd_gather(cd0_hbm)
    lax.fori_loop(0, POS_W // 16, make_fi, 0)
    w_gather(w0_hbm)
    lax.fori_loop(0, POS_W // 16, pp_set, 0)
    cd_gather(cd1_hbm)
    lax.fori_loop(0, POS_W // 16, make_fi, 0)
    w_gather(w1_hbm)
    lax.fori_loop(0, POS_W // 16, pp_add, 0)
    pltpu.sync_copy(pp_v, pp_out.at[pl.ds(pbase, POS_W)])


def _sc_gather(k01_flat, u_flat, indptr, indices, pos_flat, cd0, cd1, w0f, w1f,
               rrf):
    mesh = plsc.VectorSubcoreMesh(core_axis_name="c", subcore_axis_name="s")
    fn = pl.kernel(
        _sc_body,
        out_type=(
            jax.ShapeDtypeStruct((NEG_T,), jnp.int32),
            jax.ShapeDtypeStruct((NEG_T,), jnp.float32),
            jax.ShapeDtypeStruct((POS_T,), jnp.float32),
        ),
        mesh=mesh,
        scratch_types=[
            pltpu.VMEM((NEG_W,), jnp.int32),
            pltpu.VMEM((NEG_W,), jnp.float32),
            pltpu.VMEM((K * K + 1,), jnp.int32),
            pltpu.VMEM((NEG_CH, CH), jnp.int32),
            pltpu.VMEM((NEG_W,), jnp.int32),
            pltpu.VMEM((NEG_W,), jnp.float32),
            pltpu.VMEM((NEG_W,), jnp.float32),
            pltpu.VMEM((POS_CH, CH), jnp.int32),
            pltpu.VMEM((POS_W,), jnp.int32),
            pltpu.VMEM((POS_CH, CH), jnp.int32),
            pltpu.VMEM((POS_W,), jnp.float32),
            pltpu.VMEM((POS_W,), jnp.float32),
            pltpu.SemaphoreType.DMA,
        ],
        compiler_params=pltpu.CompilerParams(needs_layout_passes=False),
    )
    return fn(k01_flat, u_flat, indptr, indices, pos_flat, cd0, cd1, w0f, w1f,
              rrf)


def kernel(query, pos_items, c0, c1, c0_, c1_, wkk, indices, indptr, cd0, cd1):
    k01, u, rr, w0, w1 = _tc_sample(query, c0, c1, c0_, c1_, wkk)
    neg_flat, negp_flat, pp_flat = _sc_gather(
        k01.reshape(-1),
        u.reshape(-1),
        indptr.astype(jnp.int32),
        indices.astype(jnp.int32),
        pos_items.reshape(NW, POS_CH, CH).astype(jnp.int32),
        cd0.astype(jnp.int32),
        cd1.astype(jnp.int32),
        w0.reshape(-1),
        w1.reshape(-1),
        rr.reshape(-1),
    )
    pos_prob = pp_flat.reshape(B, L)
    neg_items = neg_flat.reshape(B, NN)
    neg_prob = negp_flat.reshape(B, NN)
    return (pos_prob, neg_items, neg_prob)


# speedup vs baseline: 9.8422x; 1.0184x over previous
import functools

import jax
import jax.numpy as jnp
import numpy as np
from jax import lax
from jax.experimental import pallas as pl
from jax.experimental.pallas import tpu as pltpu
from jax.experimental.pallas import tpu_sc as plsc

B = 4096
D = 64
K = 64
NN = 64
L = 20


_ROTS = ((13, 15, 26, 6), (17, 29, 16, 24))


def _np_threefry(k0, k1, x0, x1):
    def rotl(x, d):
        return ((x << np.uint32(d)) | (x >> np.uint32(32 - d))).astype(np.uint32)

    k0 = np.uint32(k0)
    k1 = np.uint32(k1)
    ks = [k0, k1, np.uint32(k0 ^ k1 ^ np.uint32(0x1BD11BDA))]
    x0 = (x0 + k0).astype(np.uint32)
    x1 = (x1 + k1).astype(np.uint32)
    for r in range(5):
        for rot in _ROTS[r % 2]:
            x0 = (x0 + x1).astype(np.uint32)
            x1 = rotl(x1, rot)
            x1 = (x1 ^ x0).astype(np.uint32)
        x0 = (x0 + ks[(r + 1) % 3]).astype(np.uint32)
        x1 = (x1 + ks[(r + 2) % 3] + np.uint32(r + 1)).astype(np.uint32)
    return x0, x1


def _fold(key, data):
    o0, o1 = _np_threefry(key[0], key[1], np.uint32([0]), np.uint32([data]))
    return (int(o0[0]), int(o1[0]))


_BASE_KEY = (0, 42)
_KEY_G0 = _fold(_BASE_KEY, 0)
_KEY_G1 = _fold(_BASE_KEY, 1)
_KEY_U = _fold(_BASE_KEY, 2)

_TINY = np.float32(np.finfo(np.float32).tiny)


def _i32(x):
    return np.int32(np.uint32(x & 0xFFFFFFFF).view(np.int32))


def _tf_bits(key, ctr):
    k0 = _i32(key[0])
    k1 = _i32(key[1])
    ks2 = _i32(key[0] ^ key[1] ^ 0x1BD11BDA)
    ks = [k0, k1, ks2]

    def rotl(x, d):
        return lax.shift_left(x, np.int32(d)) | lax.shift_right_logical(
            x, np.int32(32 - d)
        )

    x0 = jnp.full_like(ctr, k0)
    x1 = ctr + k1
    for r in range(5):
        for rot in _ROTS[r % 2]:
            x0 = x0 + x1
            x1 = rotl(x1, rot)
            x1 = x1 ^ x0
        x0 = x0 + ks[(r + 1) % 3]
        x1 = x1 + ks[(r + 2) % 3] + np.int32(r + 1)
    return x0 ^ x1


def _bits_to_u01(bits):
    fb = lax.shift_right_logical(bits, np.int32(9)) | np.int32(0x3F800000)
    return lax.bitcast_convert_type(fb, jnp.float32) - np.float32(1.0)


def _gumbel_from_bits(bits):
    u = jnp.maximum(_bits_to_u01(bits), _TINY)
    return -jnp.log(-jnp.log(u))


R = 128
NLANES = R * NN
GRID = B // R


def _dot(a, b, dims, precision=None):
    return lax.dot_general(a, b, (dims, ((), ())),
                           preferred_element_type=jnp.float32,
                           precision=precision)


def _dot_x(a, b, dims):
    return _dot(a, b, dims, precision=lax.Precision.HIGHEST)


def _tc_body(q_ref, c0_ref, c1_ref, c0p_ref, c1p_ref, wkk_ref,
             k01_ref, u_ref, rr_ref, w0_ref, w1_ref):
    i = pl.program_id(0)
    q = q_ref[...]
    q0 = q[:, :32]
    q1 = q[:, 32:]
    r0 = _dot(q0, c0_ref[...], ((1,), (1,)))
    r1 = _dot(q1, c1_ref[...], ((1,), (1,)))

    def softmax(x):
        m = jnp.max(x, axis=-1, keepdims=True)
        e = jnp.exp(x - m)
        return e / jnp.sum(e, axis=-1, keepdims=True)

    r0s = softmax(r0)
    r1s = softmax(r1)
    wkk = wkk_ref[...]
    s0 = _dot(r1s, wkk, ((1,), (1,))) * r0s
    logits0 = jnp.where(s0 > 0, jnp.log(jnp.maximum(s0, 1e-30)), -1e30)
    logits0T = logits0.T
    r1sT = r1s.T
    rr_ref[...] = jnp.concatenate([r0, r1], axis=1)

    lane_b = lax.shift_right_logical(
        lax.broadcasted_iota(jnp.int32, (R, NLANES), 1), np.int32(6)
    )
    row_r = lax.broadcasted_iota(jnp.int32, (R, NLANES), 0)
    E = jnp.where(lane_b == row_r, np.float32(1.0), np.float32(0.0))

    Ltile = _dot_x(logits0T, E, ((1,), (0,)))
    r1sTile = _dot_x(r1sT, E, ((1,), (0,)))

    lane_row = lax.broadcasted_iota(jnp.int32, (1, NLANES), 1)
    kio_col = lax.broadcasted_iota(jnp.int32, (K, 1), 0)
    kio = lax.broadcasted_iota(jnp.int32, (K, NLANES), 0)
    ctr = (kio_col + i * np.int32(NLANES * K)) + lax.shift_left(lane_row, np.int32(6))

    g0 = _gumbel_from_bits(_tf_bits(_KEY_G0, ctr))
    t0 = Ltile + g0
    m0 = jnp.max(t0, axis=0, keepdims=True)
    k0 = jnp.min(jnp.where(t0 == m0, kio, np.int32(K)), axis=0, keepdims=True)

    oh0 = jnp.where(kio == k0, np.float32(1.0), np.float32(0.0))
    subwkkT = _dot_x(wkk, oh0, ((0,), (0,)))
    s1T = subwkkT * r1sTile
    g1 = _gumbel_from_bits(_tf_bits(_KEY_G1, ctr))
    t1 = jnp.where(s1T > 0, jnp.log(jnp.maximum(s1T, 1e-30)), -1e30) + g1
    m1 = jnp.max(t1, axis=0, keepdims=True)
    k1 = jnp.min(jnp.where(t1 == m1, kio, np.int32(K)), axis=0, keepdims=True)
    k01_ref[...] = k0 * np.int32(K) + k1

    ctr2 = i * np.int32(NLANES) + lax.broadcasted_iota(jnp.int32, (1, NLANES), 1)
    u01 = _bits_to_u01(_tf_bits(_KEY_U, ctr2))
    u_ref[...] = jnp.maximum(np.float32(0.0), u01)

    w0_ref[...] = _dot(q0, c0p_ref[...], ((1,), (1,)))
    w1_ref[...] = _dot(q1, c1p_ref[...], ((1,), (1,)))


def _tc_sample(query, c0, c1, c0_, c1_, wkk):
    full = lambda s: pl.BlockSpec(s, lambda i: (0, 0))
    return pl.pallas_call(
        _tc_body,
        grid=(GRID,),
        in_specs=[
            pl.BlockSpec((R, D), lambda i: (i, 0)),
            full((K, 32)),
            full((K, 32)),
            full((2 * K, 32)),
            full((2 * K, 32)),
            full((K, K)),
        ],
        out_specs=[
            pl.BlockSpec((1, NLANES), lambda i: (0, i)),
            pl.BlockSpec((1, NLANES), lambda i: (0, i)),
            pl.BlockSpec((R, 2 * K), lambda i: (i, 0)),
            pl.BlockSpec((R, 2 * K), lambda i: (i, 0)),
            pl.BlockSpec((R, 2 * K), lambda i: (i, 0)),
        ],
        out_shape=[
            jax.ShapeDtypeStruct((1, B * NN), jnp.int32),
            jax.ShapeDtypeStruct((1, B * NN), jnp.float32),
            jax.ShapeDtypeStruct((B, 2 * K), jnp.float32),
            jax.ShapeDtypeStruct((B, 2 * K), jnp.float32),
            jax.ShapeDtypeStruct((B, 2 * K), jnp.float32),
        ],
    )(query, c0, c1,
      jnp.pad(c0_, ((0, K - 1), (0, 0))),
      jnp.pad(c1_, ((0, K - 1), (0, 0))),
      wkk)


NC = 2
NS = 16
NW = NC * NS
NEG_T = B * NN
POS_T = B * L
NEG_W = NEG_T // NW
POS_W = POS_T // NW
CH = 128
NEG_CH = NEG_W // CH
POS_CH = POS_W // CH
VPC = CH // 16


def _sc_body(k01_hbm, u_hbm, iptr_hbm, items_hbm, pos_hbm, cd0_hbm, cd1_hbm,
             w0_hbm, w1_hbm, rr_hbm,
             neg_out, negp_out, pp_out,
             k01_v, u_v, iptr_v, idx_v, rows_v, p0r_v, p1r_v,
             pi_v, kp_v, fi_v, wr_v, pp_v,
             sem):
    wid = lax.axis_index("s") * NC + lax.axis_index("c")
    nbase = wid * NEG_W

    pltpu.sync_copy(k01_hbm.at[pl.ds(nbase, NEG_W)], k01_v)
    pltpu.sync_copy(u_hbm.at[pl.ds(nbase, NEG_W)], u_v)
    pltpu.sync_copy(iptr_hbm, iptr_v)

    def neg_pos_body(j, carry):
        sl = pl.ds(j * 16, 16)
        idx = k01_v[sl]
        s = plsc.load_gather(iptr_v, [idx])
        e = plsc.load_gather(iptr_v, [idx + 1])
        ii = ((e - s).astype(jnp.float32) * u_v[sl]).astype(jnp.int32)
        idx_v[j // VPC, pl.ds((j % VPC) * 16, 16)] = ii + s
        return carry

    lax.fori_loop(0, NEG_W // 16, neg_pos_body, 0)

    def neg_fire(j, carry):
        pltpu.async_copy(items_hbm.at[idx_v.at[j]], rows_v.at[pl.ds(j * CH, CH)], sem)
        return carry

    lax.fori_loop(0, NEG_CH, neg_fire, 0)
    pltpu.make_async_copy(items_hbm.at[pl.ds(0, NEG_W)], rows_v, sem).wait()

    def neg_add1(j, carry):
        sl = pl.ds(j * 16, 16)
        rows_v[sl] = rows_v[sl] + 1
        return carry

    lax.fori_loop(0, NEG_W // 16, neg_add1, 0)
    pltpu.sync_copy(rows_v, neg_out.at[pl.ds(nbase, NEG_W)])

    def fi0_body(j, carry):
        sl = pl.ds(j * 16, 16)
        bn = nbase + j * 16 + lax.iota(jnp.int32, 16)
        b = lax.shift_right_logical(bn, 6)
        k0 = lax.shift_right_logical(k01_v[sl], 6)
        idx_v[j // VPC, pl.ds((j % VPC) * 16, 16)] = b * np.int32(128) + k0
        return carry

    def fi1_body(j, carry):
        sl = pl.ds(j * 16, 16)
        bn = nbase + j * 16 + lax.iota(jnp.int32, 16)
        b = lax.shift_right_logical(bn, 6)
        k1 = k01_v[sl] & np.int32(63)
        idx_v[j // VPC, pl.ds((j % VPC) * 16, 16)] = b * np.int32(128) + np.int32(K) + k1
        return carry

    def rr_fire(dst_v):
        def fire(j, carry):
            pltpu.async_copy(rr_hbm.at[idx_v.at[j]], dst_v.at[pl.ds(j * CH, CH)], sem)
            return carry
        lax.fori_loop(0, NEG_CH, fire, 0)
        pltpu.make_async_copy(rr_hbm.at[pl.ds(0, NEG_W)], dst_v, sem).wait()

    lax.fori_loop(0, NEG_W // 16, fi0_body, 0)
    rr_fire(p0r_v)
    lax.fori_loop(0, NEG_W // 16, fi1_body, 0)
    rr_fire(p1r_v)

    def negp_body(j, carry):
        sl = pl.ds(j * 16, 16)
        p0r_v[sl] = p0r_v[sl] + p1r_v[sl]
        return carry

    lax.fori_loop(0, NEG_W // 16, negp_body, 0)
    pltpu.sync_copy(p0r_v, negp_out.at[pl.ds(nbase, NEG_W)])

    pbase = wid * POS_W
    pltpu.sync_copy(pos_hbm.at[wid], pi_v)

    def cd_gather(cd_hbm):
        def fire(j, carry):
            pltpu.async_copy(cd_hbm.at[pi_v.at[j]], kp_v.at[pl.ds(j * CH, CH)], sem)
            return carry
        lax.fori_loop(0, POS_CH, fire, 0)
        pltpu.make_async_copy(cd_hbm.at[pl.ds(0, POS_W)], kp_v, sem).wait()

    def w_gather(w_hbm):
        def fire(j, carry):
            pltpu.async_copy(w_hbm.at[fi_v.at[j]], wr_v.at[pl.ds(j * CH, CH)], sem)
            return carry
        lax.fori_loop(0, POS_CH, fire, 0)
        pltpu.make_async_copy(w_hbm.at[pl.ds(0, POS_W)], wr_v, sem).wait()

    def make_fi(j, carry):
        sl = pl.ds(j * 16, 16)
        gi = pbase + j * 16 + lax.iota(jnp.int32, 16)
        b = lax.div(gi, np.int32(L))
        fi_v[j // VPC, pl.ds((j % VPC) * 16, 16)] = b * np.int32(128) + kp_v[sl]
        return carry

    def pp_set(j, carry):
        sl = pl.ds(j * 16, 16)
        pp_v[sl] = wr_v[sl]
        return carry

    def pp_add(j, carry):
        sl = pl.ds(j * 16, 16)
        pp_v[sl] = pp_v[sl] + wr_v[sl]
        return carry

    cd_gather(cd0_hbm)
    lax.fori_loop(0, POS_W // 16, make_fi, 0)
    w_gather(w0_hbm)
    lax.fori_loop(0, POS_W // 16, pp_set, 0)
    cd_gather(cd1_hbm)
    lax.fori_loop(0, POS_W // 16, make_fi, 0)
    w_gather(w1_hbm)
    lax.fori_loop(0, POS_W // 16, pp_add, 0)
    pltpu.sync_copy(pp_v, pp_out.at[pl.ds(pbase, POS_W)])


def _sc_gather(k01_flat, u_flat, indptr, indices, pos_flat, cd0, cd1, w0f, w1f,
               rrf):
    mesh = plsc.VectorSubcoreMesh(core_axis_name="c", subcore_axis_name="s")
    fn = pl.kernel(
        _sc_body,
        out_type=(
            jax.ShapeDtypeStruct((NEG_T,), jnp.int32),
            jax.ShapeDtypeStruct((NEG_T,), jnp.float32),
            jax.ShapeDtypeStruct((POS_T,), jnp.float32),
        ),
        mesh=mesh,
        scratch_types=[
            pltpu.VMEM((NEG_W,), jnp.int32),
            pltpu.VMEM((NEG_W,), jnp.float32),
            pltpu.VMEM((K * K + 1,), jnp.int32),
            pltpu.VMEM((NEG_CH, CH), jnp.int32),
            pltpu.VMEM((NEG_W,), jnp.int32),
            pltpu.VMEM((NEG_W,), jnp.float32),
            pltpu.VMEM((NEG_W,), jnp.float32),
            pltpu.VMEM((POS_CH, CH), jnp.int32),
            pltpu.VMEM((POS_W,), jnp.int32),
            pltpu.VMEM((POS_CH, CH), jnp.int32),
            pltpu.VMEM((POS_W,), jnp.float32),
            pltpu.VMEM((POS_W,), jnp.float32),
            pltpu.SemaphoreType.DMA,
        ],
        compiler_params=pltpu.CompilerParams(needs_layout_passes=False),
    )
    return fn(k01_flat, u_flat, indptr, indices, pos_flat, cd0, cd1, w0f, w1f,
              rrf)


def kernel(query, pos_items, c0, c1, c0_, c1_, wkk, indices, indptr, cd0, cd1):
    k01, u, rr, w0, w1 = _tc_sample(query, c0, c1, c0_, c1_, wkk)
    neg_flat, negp_flat, pp_flat = _sc_gather(
        k01.reshape(-1),
        u.reshape(-1),
        indptr.astype(jnp.int32),
        indices.astype(jnp.int32),
        pos_items.reshape(NW, POS_CH, CH).astype(jnp.int32),
        cd0.astype(jnp.int32),
        cd1.astype(jnp.int32),
        w0.reshape(-1),
        w1.reshape(-1),
        rr.reshape(-1),
    )
    pos_prob = pp_flat.reshape(B, L)
    neg_items = neg_flat.reshape(B, NN)
    neg_prob = negp_flat.reshape(B, NN)
    return (pos_prob, neg_items, neg_prob)
